# baseline (speedup 1.0000x reference)
import functools

import jax
import jax.numpy as jnp
from jax import lax
from jax.experimental import pallas as pl
from jax.experimental.pallas import tpu as pltpu
from jax.experimental.pallas import tpu_sc as plsc

NC = 2
NS = 16
LANES = 16
EB = 128


def _fill(ref, rows, cols, val):
    v = jnp.full((LANES,), val, dtype=ref.dtype)

    def row_body(r, c):
        def col_body(k, c2):
            ref[r, pl.ds(k * LANES, LANES)] = v
            return c2
        return lax.fori_loop(0, cols // LANES, col_body, c)

    lax.fori_loop(0, rows, row_body, 0)


def _zero_spmem_rows(zsrc, dst_sh, base, total_rows):
    zr = zsrc.shape[0]
    off = 0
    while off < total_rows:
        ch = min(zr, total_rows - off)
        pltpu.sync_copy(zsrc.at[pl.ds(0, ch)], dst_sh.at[pl.ds(base + off, ch)])
        off += ch


def _make_sc_aggregate(n_nodes, n_blocks, n_parts, with_deg):
    ppc = n_parts // NC
    bpt = n_blocks // NS
    n_pad = ((n_nodes + 1 + NS - 1) // NS) * NS
    zrows = n_pad // NS
    lrows = n_nodes - (NS - 1) * zrows
    assert 0 < lrows <= zrows

    out_type = [jax.ShapeDtypeStruct((n_parts, n_nodes, 128), jnp.float32)]
    if with_deg:
        out_type.append(jax.ShapeDtypeStruct((NC, n_nodes, 16), jnp.float32))

    cb = 8
    nbuf = 2
    n_ch = bpt // cb
    assert bpt % cb == 0 and n_ch % 2 == 0 and n_ch >= 4
    scratch = dict(
        col_v=pltpu.VMEM((2, cb, 1, EB), jnp.int32),
        row_v=pltpu.VMEM((2, cb, 1, EB), jnp.int32),
        acc_sh=pltpu.VMEM_SHARED((n_pad, 128), jnp.float32),
        sem=pltpu.SemaphoreType.DMA,
        sem_i=pltpu.SemaphoreType.DMA,
        sem_s=pltpu.SemaphoreType.DMA,
    )
    for b in range(nbuf):
        scratch["gath%d" % b] = pltpu.VMEM((EB, 128), jnp.float32)
    if with_deg:
        scratch["ones_v"] = pltpu.VMEM((EB, 16), jnp.float32)
        scratch["z16"] = pltpu.VMEM((64, 16), jnp.float32)
        scratch["deg_sh"] = pltpu.VMEM_SHARED((n_pad, 16), jnp.float32)
        scratch["sem_d"] = pltpu.SemaphoreType.DMA

    mesh = plsc.VectorSubcoreMesh(
        core_axis_name="c", subcore_axis_name="s",
        num_cores=NC, num_subcores=NS)

    @functools.partial(pl.kernel, out_type=tuple(out_type), mesh=mesh,
                       scratch_types=scratch,
                       compiler_params=pltpu.CompilerParams(
                           use_tc_tiling_on_sc=False))
    def sc_agg(src_hbm, colb_hbm, rowb_hbm, agg_out, *rest, **scr):
        if with_deg:
            deg_out = rest[0]
        col_v = scr["col_v"]; row_v = scr["row_v"]
        acc_sh = scr["acc_sh"]; sem = scr["sem"]; sem_i = scr["sem_i"]
        sem_s = scr["sem_s"]
        bufs = [scr["gath%d" % b] for b in range(nbuf)]

        cid = lax.axis_index("c")
        sid = lax.axis_index("s")

        blk0 = sid * bpt

        if with_deg:
            _fill(scr["ones_v"], EB, 16, 1.0)
            _fill(scr["z16"], 64, 16, 0.0)
            _zero_spmem_rows(scr["z16"], scr["deg_sh"], sid * zrows, zrows)

        def idx_load(q, c0, bset):
            pltpu.async_copy(
                colb_hbm.at[q].at[pl.ds(c0, cb)], col_v.at[bset], sem_i)
            pltpu.async_copy(
                rowb_hbm.at[pl.ds(c0, cb)], row_v.at[bset], sem_i)

        def idx_wait():
            pltpu.make_async_copy(
                rowb_hbm.at[pl.ds(0, cb)], col_v.at[0], sem_i).wait()
            pltpu.make_async_copy(
                rowb_hbm.at[pl.ds(0, cb)], row_v.at[0], sem_i).wait()

        def g_wait():
            pltpu.make_async_copy(
                src_hbm.at[col_v.at[0].at[0, 0]], bufs[0], sem).wait()

        def s_wait():
            pltpu.make_async_copy(
                bufs[0], acc_sh.at[row_v.at[0].at[0, 0]], sem_s).wait()

        def d_wait():
            pltpu.make_async_copy(
                scr["ones_v"], scr["deg_sh"].at[row_v.at[0].at[0, 0]],
                scr["sem_d"]).wait()

        for p in range(ppc):
            q = cid * ppc + p
            _fill(bufs[0], EB, 128, 0.0)
            _zero_spmem_rows(bufs[0], acc_sh, sid * zrows, zrows)
            plsc.subcore_barrier()

            do_deg = with_deg and p == 0

            def run_chunk(ci, bset, first, clamp):
                if clamp:
                    nci = jnp.minimum(ci + 1, n_ch - 2)
                else:
                    nci = ci + 1
                idx_load(q, blk0 + nci * cb, 1 - bset)

                cv = col_v.at[bset]
                rv = row_v.at[bset]
                cvn = col_v.at[1 - bset]

                if first:
                    pltpu.async_copy(src_hbm.at[cv.at[0, 0]], bufs[0], sem)

                if do_deg:
                    deg_here = (ci < n_ch // 2) == (cid == 0)

                for k in range(cb):
                    if not (first and k < 1):
                        s_wait()
                    if k == cb - 2:
                        idx_wait()
                    if k + 1 < cb:
                        pltpu.async_copy(src_hbm.at[cv.at[k + 1, 0]],
                                         bufs[(k + 1) % nbuf], sem)
                    else:
                        pltpu.async_copy(src_hbm.at[cvn.at[k + 1 - cb, 0]],
                                         bufs[(k + 1) % nbuf], sem)
                    g_wait()
                    pltpu.async_copy(bufs[k % nbuf],
                                     acc_sh.at[rv.at[k, 0]], sem_s,
                                     add=True)
                    if do_deg:
                        @pl.when(deg_here)
                        def _():
                            pltpu.async_copy(
                                scr["ones_v"],
                                scr["deg_sh"].at[rv.at[k, 0]],
                                scr["sem_d"], add=True)

            idx_load(q, blk0, 0)
            idx_wait()
            run_chunk(0, 0, True, False)
            run_chunk(1, 1, False, False)

            def chunk2(ci2, carry):
                run_chunk(2 * ci2, 0, False, False)
                run_chunk(2 * ci2 + 1, 1, False, True)
                return carry

            lax.fori_loop(1, n_ch // 2, chunk2, 0)

            g_wait()
            s_wait()
            if do_deg:
                def ddrain(i, carry):
                    d_wait()
                    return carry
                lax.fori_loop(0, bpt // 2, ddrain, 0)
            plsc.subcore_barrier()

            r0 = sid * zrows

            @pl.when(sid < NS - 1)
            def _():
                pltpu.sync_copy(acc_sh.at[pl.ds(r0, zrows)],
                                agg_out.at[q].at[pl.ds(r0, zrows)])

            @pl.when(sid == NS - 1)
            def _():
                pltpu.sync_copy(acc_sh.at[pl.ds(r0, lrows)],
                                agg_out.at[q].at[pl.ds(r0, lrows)])

            if with_deg and p == 0:
                @pl.when(sid < NS - 1)
                def _():
                    pltpu.sync_copy(scr["deg_sh"].at[pl.ds(r0, zrows)],
                                    deg_out.at[cid].at[pl.ds(r0, zrows)])

                @pl.when(sid == NS - 1)
                def _():
                    pltpu.sync_copy(scr["deg_sh"].at[pl.ds(r0, lrows)],
                                    deg_out.at[cid].at[pl.ds(r0, lrows)])
            plsc.subcore_barrier()

    return sc_agg


def _dot(a, b):
    return jnp.dot(a, b, precision=lax.Precision.DEFAULT,
                   preferred_element_type=jnp.float32)


def _make_tc_root(n_nodes, n_parts, rblk):
    grid = (n_nodes // rblk,)

    def body(x_ref, wr_ref, b_ref, r_ref):
        s = _dot(x_ref[0], wr_ref[0])
        for p in range(1, n_parts):
            s = s + _dot(x_ref[p], wr_ref[p])
        r_ref[...] = s + b_ref[0:1, :]

    return pl.pallas_call(
        body,
        grid=grid,
        in_specs=[
            pl.BlockSpec((n_parts, rblk, 128), lambda i: (0, i, 0)),
            pl.BlockSpec((n_parts, 128, 512), lambda i: (0, 0, 0)),
            pl.BlockSpec((8, 512), lambda i: (0, 0)),
        ],
        out_specs=pl.BlockSpec((rblk, 512), lambda i: (i, 0)),
        out_shape=jax.ShapeDtypeStruct((n_nodes, 512), jnp.float32),
    )


def _make_tc_matmul(n_nodes, n_parts, rblk):
    grid = (n_nodes // rblk,)

    def body(agg_ref, deg_ref, r_ref, wl_ref, t_ref, st_ref):
        i = pl.program_id(0)
        deg = deg_ref[0, :, 0:1] + deg_ref[1, :, 0:1]
        rdeg = 1.0 / jnp.maximum(deg, 1.0)
        s_l = _dot(agg_ref[0], wl_ref[0])
        for p in range(1, n_parts):
            s_l = s_l + _dot(agg_ref[p], wl_ref[p])
        t = rdeg * s_l + r_ref[...]
        t_ref[...] = t

        @pl.when(i == 0)
        def _():
            st_ref[...] = jnp.zeros_like(st_ref)
        st_ref[0:1, :] += jnp.sum(t, axis=0, keepdims=True)
        st_ref[1:2, :] += jnp.sum(t * t, axis=0, keepdims=True)

    return pl.pallas_call(
        body,
        grid=grid,
        in_specs=[
            pl.BlockSpec((n_parts, rblk, 128), lambda i: (0, i, 0)),
            pl.BlockSpec((2, rblk, 16), lambda i: (0, i, 0)),
            pl.BlockSpec((rblk, 512), lambda i: (i, 0)),
            pl.BlockSpec((n_parts, 128, 512), lambda i: (0, 0, 0)),
        ],
        out_specs=[
            pl.BlockSpec((rblk, 512), lambda i: (i, 0)),
            pl.BlockSpec((8, 512), lambda i: (0, 0)),
        ],
        out_shape=[
            jax.ShapeDtypeStruct((n_nodes, 512), jnp.float32),
            jax.ShapeDtypeStruct((8, 512), jnp.float32),
        ],
    )


def _make_tc_norm(n_nodes, rblk, out_parts):
    grid = (n_nodes // rblk,)
    inv_n = 1.0 / n_nodes

    def body(t_ref, st_ref, g_ref, be_ref, out_ref):
        t = t_ref[...]
        mu = st_ref[0:1, :] * inv_n
        var = st_ref[1:2, :] * inv_n - mu * mu
        scale = g_ref[0:1, :] * lax.rsqrt(var + 1e-5)
        h = (t - mu) * scale + be_ref[0:1, :]
        h = jnp.where(h > 0, h, jnp.exp(jnp.minimum(h, 0.0)) - 1.0)
        if out_parts:
            for q in range(4):
                out_ref[q] = h[:, q * 128:(q + 1) * 128]
        else:
            out_ref[...] = h

    if out_parts:
        out_spec = pl.BlockSpec((4, rblk, 128), lambda i: (0, i, 0))
        out_shape = jax.ShapeDtypeStruct((4, n_nodes, 128), jnp.float32)
    else:
        out_spec = pl.BlockSpec((rblk, 512), lambda i: (i, 0))
        out_shape = jax.ShapeDtypeStruct((n_nodes, 512), jnp.float32)

    return pl.pallas_call(
        body,
        grid=grid,
        in_specs=[
            pl.BlockSpec((rblk, 512), lambda i: (i, 0)),
            pl.BlockSpec((8, 512), lambda i: (0, 0)),
            pl.BlockSpec((1, 512), lambda i: (0, 0)),
            pl.BlockSpec((1, 512), lambda i: (0, 0)),
        ],
        out_specs=out_spec,
        out_shape=out_shape,
    )


def kernel(x, edge_index, W1l, b1l, W1r, g1, be1, W2l, b2l, W2r, g2, be2):
    n, d_in = x.shape
    e = edge_index.shape[1]
    d_h = W1l.shape[0]
    d_out = W2l.shape[0]
    p1 = d_in // 128
    p2 = d_h // 128
    rblk = 400

    row = edge_index[0]
    col = edge_index[1]

    nb = ((e + 8 * NS * EB - 1) // (8 * NS * EB)) * (8 * NS)
    e_pad = nb * EB
    row_p = jnp.concatenate([row, jnp.full((e_pad - e,), n, jnp.int32)])
    col_p = jnp.concatenate([col, jnp.zeros((e_pad - e,), jnp.int32)])
    rowb = row_p.reshape(nb, 1, EB)
    col2 = (col_p[None, :] + (jnp.arange(p1, dtype=jnp.int32) * n)[:, None]
            ).reshape(p1, nb, 1, EB)
    col4 = (col_p[None, :] + (jnp.arange(p2, dtype=jnp.int32) * n)[:, None]
            ).reshape(p2, nb, 1, EB)

    x2 = x.reshape(n, p1, 128).transpose(1, 0, 2)

    w1l_t = W1l.T.reshape(p1, 128, d_h)
    w1r_t = W1r.T.reshape(p1, 128, d_h)
    w2l_t = W2l.T.reshape(p2, 128, d_out)
    w2r_t = W2r.T.reshape(p2, 128, d_out)
    b1 = jnp.broadcast_to(b1l[None, :], (8, d_h))
    b2 = jnp.broadcast_to(b2l[None, :], (8, d_out))
    g1_2 = g1[None, :]
    be1_2 = be1[None, :]
    g2_2 = g2[None, :]
    be2_2 = be2[None, :]

    root1 = _make_tc_root(n, p1, rblk)
    r1 = root1(x2, w1r_t, b1)
    sc1 = _make_sc_aggregate(n, nb, p1, with_deg=True)
    agg1, deg = sc1(x2.reshape(p1 * n, 128), col2, rowb)
    mm1 = _make_tc_matmul(n, p1, rblk)
    t1, st1 = mm1(agg1, deg, r1, w1l_t)
    norm1 = _make_tc_norm(n, rblk, out_parts=True)
    h4 = norm1(t1, st1, g1_2, be1_2)

    root2 = _make_tc_root(n, p2, rblk)
    r2 = root2(h4, w2r_t, b2)
    sc2 = _make_sc_aggregate(n, nb, p2, with_deg=False)
    (agg2,) = sc2(h4.reshape(p2 * n, 128), col4, rowb)
    mm2 = _make_tc_matmul(n, p2, rblk)
    t2, st2 = mm2(agg2, deg, r2, w2l_t)
    norm2 = _make_tc_norm(n, rblk, out_parts=False)
    out = norm2(t2, st2, g2_2, be2_2)
    return out


# speedup vs baseline: 1.5248x; 1.5248x over previous
import functools

import jax
import jax.numpy as jnp
from jax import lax
from jax.experimental import pallas as pl
from jax.experimental.pallas import tpu as pltpu
from jax.experimental.pallas import tpu_sc as plsc

NC = 2
NS = 16
LANES = 16
EB = 128


def _fill(ref, rows, cols, val):
    w = LANES * 4 // ref.dtype.itemsize
    v = jnp.full((w,), val, dtype=ref.dtype)

    def row_body(r, c):
        def col_body(k, c2):
            ref[r, pl.ds(k * w, w)] = v
            return c2
        return lax.fori_loop(0, cols // w, col_body, c)

    lax.fori_loop(0, rows, row_body, 0)


def _zero_spmem_rows(zsrc, dst_sh, base, total_rows):
    zr = zsrc.shape[0]
    off = 0
    while off < total_rows:
        ch = min(zr, total_rows - off)
        pltpu.sync_copy(zsrc.at[pl.ds(0, ch)], dst_sh.at[pl.ds(base + off, ch)])
        off += ch


def _make_sc_aggregate(n_nodes, n_blocks, n_parts, pw, dtype, with_deg):
    ppc = n_parts // NC
    bpt = n_blocks // NS
    n_pad = ((n_nodes + 1 + NS - 1) // NS) * NS
    zrows = n_pad // NS
    lrows = n_nodes - (NS - 1) * zrows
    assert 0 < lrows <= zrows

    out_type = [jax.ShapeDtypeStruct((n_parts, n_nodes, pw), dtype)]
    if with_deg:
        out_type.append(jax.ShapeDtypeStruct((NC, n_nodes, 16), jnp.float32))

    cb = 8
    nbuf = 2
    n_ch = bpt // cb
    assert bpt % cb == 0 and n_ch % 2 == 0 and n_ch >= 4
    scratch = dict(
        col_v=pltpu.VMEM((2, cb, 1, EB), jnp.int32),
        row_v=pltpu.VMEM((2, cb, 1, EB), jnp.int32),
        acc_sh=pltpu.VMEM_SHARED((n_pad, pw), dtype),
        sem=pltpu.SemaphoreType.DMA,
        sem_i=pltpu.SemaphoreType.DMA,
        sem_s=pltpu.SemaphoreType.DMA,
    )
    for b in range(nbuf):
        scratch["gath%d" % b] = pltpu.VMEM((EB, pw), dtype)
    if with_deg:
        scratch["ones_v"] = pltpu.VMEM((EB, 16), jnp.float32)
        scratch["z16"] = pltpu.VMEM((64, 16), jnp.float32)
        scratch["deg_sh"] = pltpu.VMEM_SHARED((n_pad, 16), jnp.float32)
        scratch["sem_d"] = pltpu.SemaphoreType.DMA

    mesh = plsc.VectorSubcoreMesh(
        core_axis_name="c", subcore_axis_name="s",
        num_cores=NC, num_subcores=NS)

    @functools.partial(pl.kernel, out_type=tuple(out_type), mesh=mesh,
                       scratch_types=scratch,
                       compiler_params=pltpu.CompilerParams(
                           use_tc_tiling_on_sc=False))
    def sc_agg(src_hbm, colb_hbm, rowb_hbm, agg_out, *rest, **scr):
        if with_deg:
            deg_out = rest[0]
        col_v = scr["col_v"]; row_v = scr["row_v"]
        acc_sh = scr["acc_sh"]; sem = scr["sem"]; sem_i = scr["sem_i"]
        sem_s = scr["sem_s"]
        bufs = [scr["gath%d" % b] for b in range(nbuf)]

        cid = lax.axis_index("c")
        sid = lax.axis_index("s")

        blk0 = sid * bpt

        if with_deg:
            _fill(scr["ones_v"], EB, 16, 1.0)
            _fill(scr["z16"], 64, 16, 0.0)
            _zero_spmem_rows(scr["z16"], scr["deg_sh"], sid * zrows, zrows)

        def idx_load(q, c0, bset):
            pltpu.async_copy(
                colb_hbm.at[q].at[pl.ds(c0, cb)], col_v.at[bset], sem_i)
            pltpu.async_copy(
                rowb_hbm.at[pl.ds(c0, cb)], row_v.at[bset], sem_i)

        def idx_wait():
            pltpu.make_async_copy(
                rowb_hbm.at[pl.ds(0, cb)], col_v.at[0], sem_i).wait()
            pltpu.make_async_copy(
                rowb_hbm.at[pl.ds(0, cb)], row_v.at[0], sem_i).wait()

        def g_wait():
            pltpu.make_async_copy(
                src_hbm.at[col_v.at[0].at[0, 0]], bufs[0], sem).wait()

        def s_wait():
            pltpu.make_async_copy(
                bufs[0], acc_sh.at[row_v.at[0].at[0, 0]], sem_s).wait()

        def d_wait():
            pltpu.make_async_copy(
                scr["ones_v"], scr["deg_sh"].at[row_v.at[0].at[0, 0]],
                scr["sem_d"]).wait()

        for p in range(ppc):
            q = cid * ppc + p
            _fill(bufs[0], EB, pw, 0.0)
            _zero_spmem_rows(bufs[0], acc_sh, sid * zrows, zrows)
            plsc.subcore_barrier()

            do_deg = with_deg and p == 0

            def run_chunk(ci, bset, first, clamp):
                if clamp:
                    nci = jnp.minimum(ci + 1, n_ch - 2)
                else:
                    nci = ci + 1
                idx_load(q, blk0 + nci * cb, 1 - bset)

                cv = col_v.at[bset]
                rv = row_v.at[bset]
                cvn = col_v.at[1 - bset]

                if first:
                    pltpu.async_copy(src_hbm.at[cv.at[0, 0]], bufs[0], sem)

                if do_deg:
                    deg_here = (ci < n_ch // 2) == (cid == 0)

                for k in range(cb):
                    if not (first and k < 1):
                        s_wait()
                    if k == cb - 2:
                        idx_wait()
                    if k + 1 < cb:
                        pltpu.async_copy(src_hbm.at[cv.at[k + 1, 0]],
                                         bufs[(k + 1) % nbuf], sem)
                    else:
                        pltpu.async_copy(src_hbm.at[cvn.at[k + 1 - cb, 0]],
                                         bufs[(k + 1) % nbuf], sem)
                    g_wait()
                    pltpu.async_copy(bufs[k % nbuf],
                                     acc_sh.at[rv.at[k, 0]], sem_s,
                                     add=True)
                    if do_deg:
                        @pl.when(deg_here)
                        def _():
                            pltpu.async_copy(
                                scr["ones_v"],
                                scr["deg_sh"].at[rv.at[k, 0]],
                                scr["sem_d"], add=True)

            idx_load(q, blk0, 0)
            idx_wait()
            run_chunk(0, 0, True, False)
            run_chunk(1, 1, False, False)

            def chunk2(ci2, carry):
                run_chunk(2 * ci2, 0, False, False)
                run_chunk(2 * ci2 + 1, 1, False, True)
                return carry

            lax.fori_loop(1, n_ch // 2, chunk2, 0)

            g_wait()
            s_wait()
            if do_deg:
                def ddrain(i, carry):
                    d_wait()
                    return carry
                lax.fori_loop(0, bpt // 2, ddrain, 0)
            plsc.subcore_barrier()

            r0 = sid * zrows

            @pl.when(sid < NS - 1)
            def _():
                pltpu.sync_copy(acc_sh.at[pl.ds(r0, zrows)],
                                agg_out.at[q].at[pl.ds(r0, zrows)])

            @pl.when(sid == NS - 1)
            def _():
                pltpu.sync_copy(acc_sh.at[pl.ds(r0, lrows)],
                                agg_out.at[q].at[pl.ds(r0, lrows)])

            if with_deg and p == 0:
                @pl.when(sid < NS - 1)
                def _():
                    pltpu.sync_copy(scr["deg_sh"].at[pl.ds(r0, zrows)],
                                    deg_out.at[cid].at[pl.ds(r0, zrows)])

                @pl.when(sid == NS - 1)
                def _():
                    pltpu.sync_copy(scr["deg_sh"].at[pl.ds(r0, lrows)],
                                    deg_out.at[cid].at[pl.ds(r0, lrows)])
            plsc.subcore_barrier()

    return sc_agg


def _dot(a, b):
    return jnp.dot(a, b, precision=lax.Precision.DEFAULT,
                   preferred_element_type=jnp.float32)


def _make_tc_matmul(n_nodes, n_parts, rblk):
    grid = (n_nodes // rblk,)

    def body(agg_ref, deg_ref, x_ref, wl_ref, wr_ref, b_ref, t_ref, st_ref):
        i = pl.program_id(0)
        deg = deg_ref[0, :, 0:1] + deg_ref[1, :, 0:1]
        rdeg = 1.0 / jnp.maximum(deg, 1.0)
        s_l = _dot(agg_ref[0].astype(jnp.float32), wl_ref[0])
        s_r = _dot(x_ref[0].astype(jnp.float32), wr_ref[0])
        for p in range(1, n_parts):
            s_l = s_l + _dot(agg_ref[p].astype(jnp.float32), wl_ref[p])
            s_r = s_r + _dot(x_ref[p].astype(jnp.float32), wr_ref[p])
        t = rdeg * s_l + s_r + b_ref[0:1, :]
        t_ref[...] = t

        @pl.when(i == 0)
        def _():
            st_ref[...] = jnp.zeros_like(st_ref)
        st_ref[0:1, :] += jnp.sum(t, axis=0, keepdims=True)
        st_ref[1:2, :] += jnp.sum(t * t, axis=0, keepdims=True)

    def make(pw):
        return pl.pallas_call(
            body,
            grid=grid,
            in_specs=[
                pl.BlockSpec((n_parts, rblk, pw), lambda i: (0, i, 0)),
                pl.BlockSpec((2, rblk, 16), lambda i: (0, i, 0)),
                pl.BlockSpec((n_parts, rblk, pw), lambda i: (0, i, 0)),
                pl.BlockSpec((n_parts, pw, 512), lambda i: (0, 0, 0)),
                pl.BlockSpec((n_parts, pw, 512), lambda i: (0, 0, 0)),
                pl.BlockSpec((8, 512), lambda i: (0, 0)),
            ],
            out_specs=[
                pl.BlockSpec((rblk, 512), lambda i: (i, 0)),
                pl.BlockSpec((8, 512), lambda i: (0, 0)),
            ],
            out_shape=[
                jax.ShapeDtypeStruct((n_nodes, 512), jnp.float32),
                jax.ShapeDtypeStruct((8, 512), jnp.float32),
            ],
        )

    return make


def _make_tc_norm(n_nodes, rblk, out_parts):
    grid = (n_nodes // rblk,)
    inv_n = 1.0 / n_nodes

    def body(t_ref, st_ref, g_ref, be_ref, out_ref):
        t = t_ref[...]
        mu = st_ref[0:1, :] * inv_n
        var = st_ref[1:2, :] * inv_n - mu * mu
        scale = g_ref[0:1, :] * lax.rsqrt(var + 1e-5)
        h = (t - mu) * scale + be_ref[0:1, :]
        h = jnp.where(h > 0, h, jnp.exp(jnp.minimum(h, 0.0)) - 1.0)
        if out_parts:
            hb = h.astype(jnp.bfloat16)
            for q in range(2):
                out_ref[q] = hb[:, q * 256:(q + 1) * 256]
        else:
            out_ref[...] = h

    if out_parts:
        out_spec = pl.BlockSpec((2, rblk, 256), lambda i: (0, i, 0))
        out_shape = jax.ShapeDtypeStruct((2, n_nodes, 256), jnp.bfloat16)
    else:
        out_spec = pl.BlockSpec((rblk, 512), lambda i: (i, 0))
        out_shape = jax.ShapeDtypeStruct((n_nodes, 512), jnp.float32)

    return pl.pallas_call(
        body,
        grid=grid,
        in_specs=[
            pl.BlockSpec((rblk, 512), lambda i: (i, 0)),
            pl.BlockSpec((8, 512), lambda i: (0, 0)),
            pl.BlockSpec((1, 512), lambda i: (0, 0)),
            pl.BlockSpec((1, 512), lambda i: (0, 0)),
        ],
        out_specs=out_spec,
        out_shape=out_shape,
    )


def kernel(x, edge_index, W1l, b1l, W1r, g1, be1, W2l, b2l, W2r, g2, be2):
    n, d_in = x.shape
    e = edge_index.shape[1]
    d_h = W1l.shape[0]
    d_out = W2l.shape[0]
    p1 = d_in // 128
    p2 = d_h // 256
    rblk = 400

    row = edge_index[0]
    col = edge_index[1]

    nb = ((e + 8 * NS * EB - 1) // (8 * NS * EB)) * (8 * NS)
    e_pad = nb * EB
    row_p = jnp.concatenate([row, jnp.full((e_pad - e,), n, jnp.int32)])
    col_p = jnp.concatenate([col, jnp.zeros((e_pad - e,), jnp.int32)])
    rowb = row_p.reshape(nb, 1, EB)
    col2 = (col_p[None, :] + (jnp.arange(2, dtype=jnp.int32) * n)[:, None]
            ).reshape(2, nb, 1, EB)

    x2 = x.reshape(n, p1, 128).transpose(1, 0, 2)
    x2b = x2.astype(jnp.bfloat16)

    w1l_t = W1l.T.reshape(p1, 128, d_h)
    w1r_t = W1r.T.reshape(p1, 128, d_h)
    w2l_t = W2l.T.reshape(p2, 256, d_out)
    w2r_t = W2r.T.reshape(p2, 256, d_out)
    b1 = jnp.broadcast_to(b1l[None, :], (8, d_h))
    b2 = jnp.broadcast_to(b2l[None, :], (8, d_out))
    g1_2 = g1[None, :]
    be1_2 = be1[None, :]
    g2_2 = g2[None, :]
    be2_2 = be2[None, :]

    sc1 = _make_sc_aggregate(n, nb, p1, 128, jnp.bfloat16, with_deg=True)
    agg1, deg = sc1(x2b.reshape(p1 * n, 128), col2, rowb)
    mm1 = _make_tc_matmul(n, p1, rblk)(128)
    t1, st1 = mm1(agg1, deg, x2, w1l_t, w1r_t, b1)
    norm1 = _make_tc_norm(n, rblk, out_parts=True)
    h2b = norm1(t1, st1, g1_2, be1_2)

    sc2 = _make_sc_aggregate(n, nb, p2, 256, jnp.bfloat16, with_deg=False)
    (agg2,) = sc2(h2b.reshape(p2 * n, 256), col2, rowb)
    mm2 = _make_tc_matmul(n, p2, rblk)(256)
    t2, st2 = mm2(agg2, deg, h2b, w2l_t, w2r_t, b2)
    norm2 = _make_tc_norm(n, rblk, out_parts=False)
    out = norm2(t2, st2, g2_2, be2_2)
    return out


# speedup vs baseline: 1.6239x; 1.0650x over previous
import functools

import jax
import jax.numpy as jnp
from jax import lax
from jax.experimental import pallas as pl
from jax.experimental.pallas import tpu as pltpu
from jax.experimental.pallas import tpu_sc as plsc

NC = 2
NS = 16
LANES = 16
EB = 128


def _fill(ref, rows, cols, val):
    w = LANES * 4 // ref.dtype.itemsize
    v = jnp.full((w,), val, dtype=ref.dtype)

    def row_body(r, c):
        def col_body(k, c2):
            ref[r, pl.ds(k * w, w)] = v
            return c2
        return lax.fori_loop(0, cols // w, col_body, c)

    lax.fori_loop(0, rows, row_body, 0)


def _zero_spmem_rows(zsrc, dst_sh, base, total_rows):
    zr = zsrc.shape[0]
    off = 0
    while off < total_rows:
        ch = min(zr, total_rows - off)
        pltpu.sync_copy(zsrc.at[pl.ds(0, ch)], dst_sh.at[pl.ds(base + off, ch)])
        off += ch


def _make_sc_aggregate(n_nodes, n_blocks, n_parts, pw, dtype, with_deg):
    ppc = n_parts // NC
    bpt = n_blocks // NS
    n_pad = ((n_nodes + 1 + NS - 1) // NS) * NS
    zrows = n_pad // NS
    lrows = n_nodes - (NS - 1) * zrows
    assert 0 < lrows <= zrows

    out_type = [jax.ShapeDtypeStruct((n_parts, n_nodes, pw), dtype)]
    if with_deg:
        out_type.append(jax.ShapeDtypeStruct((NC, n_nodes, 16), jnp.float32))

    cb = 8
    nbuf = 2
    n_ch = bpt // cb
    assert bpt % cb == 0 and n_ch % 2 == 0 and n_ch >= 4
    scratch = dict(
        col_v=pltpu.VMEM((2, cb, 1, EB), jnp.int32),
        row_v=pltpu.VMEM((2, cb, 1, EB), jnp.int32),
        acc_sh=pltpu.VMEM_SHARED((n_pad, pw), dtype),
        sem=pltpu.SemaphoreType.DMA,
        sem_i=pltpu.SemaphoreType.DMA,
        sem_s=pltpu.SemaphoreType.DMA,
    )
    for b in range(nbuf):
        scratch["gath%d" % b] = pltpu.VMEM((EB, pw), dtype)
    if with_deg:
        scratch["ones_v"] = pltpu.VMEM((EB, 16), jnp.float32)
        scratch["z16"] = pltpu.VMEM((64, 16), jnp.float32)
        scratch["deg_sh"] = pltpu.VMEM_SHARED((n_pad, 16), jnp.float32)
        scratch["sem_d"] = pltpu.SemaphoreType.DMA

    mesh = plsc.VectorSubcoreMesh(
        core_axis_name="c", subcore_axis_name="s",
        num_cores=NC, num_subcores=NS)

    @functools.partial(pl.kernel, out_type=tuple(out_type), mesh=mesh,
                       scratch_types=scratch,
                       compiler_params=pltpu.CompilerParams(
                           use_tc_tiling_on_sc=False))
    def sc_agg(src_hbm, colb_hbm, rowb_hbm, agg_out, *rest, **scr):
        if with_deg:
            deg_out = rest[0]
        col_v = scr["col_v"]; row_v = scr["row_v"]
        acc_sh = scr["acc_sh"]; sem = scr["sem"]; sem_i = scr["sem_i"]
        sem_s = scr["sem_s"]
        bufs = [scr["gath%d" % b] for b in range(nbuf)]

        cid = lax.axis_index("c")
        sid = lax.axis_index("s")

        blk0 = sid * bpt

        if with_deg:
            _fill(scr["ones_v"], EB, 16, 1.0)
            _fill(scr["z16"], 64, 16, 0.0)
            _zero_spmem_rows(scr["z16"], scr["deg_sh"], sid * zrows, zrows)

        def idx_load(q, c0, bset):
            pltpu.async_copy(
                colb_hbm.at[q].at[pl.ds(c0, cb)], col_v.at[bset], sem_i)
            pltpu.async_copy(
                rowb_hbm.at[pl.ds(c0, cb)], row_v.at[bset], sem_i)

        def idx_wait():
            pltpu.make_async_copy(
                rowb_hbm.at[pl.ds(0, cb)], col_v.at[0], sem_i).wait()
            pltpu.make_async_copy(
                rowb_hbm.at[pl.ds(0, cb)], row_v.at[0], sem_i).wait()

        def g_wait():
            pltpu.make_async_copy(
                src_hbm.at[col_v.at[0].at[0, 0]], bufs[0], sem).wait()

        def s_wait():
            pltpu.make_async_copy(
                bufs[0], acc_sh.at[row_v.at[0].at[0, 0]], sem_s).wait()

        def d_wait():
            pltpu.make_async_copy(
                scr["ones_v"], scr["deg_sh"].at[row_v.at[0].at[0, 0]],
                scr["sem_d"]).wait()

        for p in range(ppc):
            q = cid * ppc + p
            _fill(bufs[0], EB, pw, 0.0)
            _zero_spmem_rows(bufs[0], acc_sh, sid * zrows, zrows)
            plsc.subcore_barrier()

            do_deg = with_deg and p == 0

            def run_chunk(ci, bset, first, clamp):
                if clamp:
                    nci = jnp.minimum(ci + 1, n_ch - 2)
                else:
                    nci = ci + 1
                idx_load(q, blk0 + nci * cb, 1 - bset)

                cv = col_v.at[bset]
                rv = row_v.at[bset]
                cvn = col_v.at[1 - bset]

                if first:
                    pltpu.async_copy(src_hbm.at[cv.at[0, 0]], bufs[0], sem)

                if do_deg:
                    deg_here = (ci < n_ch // 2) == (cid == 0)

                for k in range(cb):
                    if not (first and k < 1):
                        s_wait()
                    if k == cb - 2:
                        idx_wait()
                    if k + 1 < cb:
                        pltpu.async_copy(src_hbm.at[cv.at[k + 1, 0]],
                                         bufs[(k + 1) % nbuf], sem)
                    else:
                        pltpu.async_copy(src_hbm.at[cvn.at[k + 1 - cb, 0]],
                                         bufs[(k + 1) % nbuf], sem)
                    g_wait()
                    pltpu.async_copy(bufs[k % nbuf],
                                     acc_sh.at[rv.at[k, 0]], sem_s,
                                     add=True)
                    if do_deg:
                        @pl.when(deg_here)
                        def _():
                            pltpu.async_copy(
                                scr["ones_v"],
                                scr["deg_sh"].at[rv.at[k, 0]],
                                scr["sem_d"], add=True)

            idx_load(q, blk0, 0)
            idx_wait()
            run_chunk(0, 0, True, False)
            run_chunk(1, 1, False, False)

            def chunk2(ci2, carry):
                run_chunk(2 * ci2, 0, False, False)
                run_chunk(2 * ci2 + 1, 1, False, True)
                return carry

            lax.fori_loop(1, n_ch // 2, chunk2, 0)

            g_wait()
            s_wait()
            if do_deg:
                def ddrain(i, carry):
                    d_wait()
                    return carry
                lax.fori_loop(0, bpt // 2, ddrain, 0)
            plsc.subcore_barrier()

            r0 = sid * zrows

            @pl.when(sid < NS - 1)
            def _():
                pltpu.sync_copy(acc_sh.at[pl.ds(r0, zrows)],
                                agg_out.at[q].at[pl.ds(r0, zrows)])

            @pl.when(sid == NS - 1)
            def _():
                pltpu.sync_copy(acc_sh.at[pl.ds(r0, lrows)],
                                agg_out.at[q].at[pl.ds(r0, lrows)])

            if with_deg and p == 0:
                @pl.when(sid < NS - 1)
                def _():
                    pltpu.sync_copy(scr["deg_sh"].at[pl.ds(r0, zrows)],
                                    deg_out.at[cid].at[pl.ds(r0, zrows)])

                @pl.when(sid == NS - 1)
                def _():
                    pltpu.sync_copy(scr["deg_sh"].at[pl.ds(r0, lrows)],
                                    deg_out.at[cid].at[pl.ds(r0, lrows)])
            plsc.subcore_barrier()

    return sc_agg


def _dot(a, b):
    return jnp.dot(a.astype(jnp.bfloat16), b.astype(jnp.bfloat16),
                   precision=lax.Precision.DEFAULT,
                   preferred_element_type=jnp.float32)


def _make_tc_matmul(n_nodes, pa, wa, px, wx, rblk):
    grid = (n_nodes // rblk,)

    def body(agg_ref, deg_ref, x_ref, wl_ref, wr_ref, b_ref, t_ref, st_ref):
        i = pl.program_id(0)
        deg = deg_ref[0, :, 0:1] + deg_ref[1, :, 0:1]
        rdeg = 1.0 / jnp.maximum(deg, 1.0)
        s_l = _dot(agg_ref[0], wl_ref[0])
        for p in range(1, pa):
            s_l = s_l + _dot(agg_ref[p], wl_ref[p])
        s_r = _dot(x_ref[0], wr_ref[0])
        for p in range(1, px):
            s_r = s_r + _dot(x_ref[p], wr_ref[p])
        t = rdeg * s_l + s_r + b_ref[0:1, :]
        t_ref[...] = t

        @pl.when(i == 0)
        def _():
            st_ref[...] = jnp.zeros_like(st_ref)
        st_ref[0:1, :] += jnp.sum(t, axis=0, keepdims=True)
        st_ref[1:2, :] += jnp.sum(t * t, axis=0, keepdims=True)

    return pl.pallas_call(
        body,
        grid=grid,
        in_specs=[
            pl.BlockSpec((pa, rblk, wa), lambda i: (0, i, 0)),
            pl.BlockSpec((2, rblk, 16), lambda i: (0, i, 0)),
            pl.BlockSpec((px, rblk, wx), lambda i: (0, i, 0)),
            pl.BlockSpec((pa, wa, 512), lambda i: (0, 0, 0)),
            pl.BlockSpec((px, wx, 512), lambda i: (0, 0, 0)),
            pl.BlockSpec((8, 512), lambda i: (0, 0)),
        ],
        out_specs=[
            pl.BlockSpec((rblk, 512), lambda i: (i, 0)),
            pl.BlockSpec((8, 512), lambda i: (0, 0)),
        ],
        out_shape=[
            jax.ShapeDtypeStruct((n_nodes, 512), jnp.float32),
            jax.ShapeDtypeStruct((8, 512), jnp.float32),
        ],
    )


def _make_tc_norm(n_nodes, rblk, out_parts):
    grid = (n_nodes // rblk,)
    inv_n = 1.0 / n_nodes

    def body(t_ref, st_ref, g_ref, be_ref, out_ref):
        t = t_ref[...]
        mu = st_ref[0:1, :] * inv_n
        var = st_ref[1:2, :] * inv_n - mu * mu
        scale = g_ref[0:1, :] * lax.rsqrt(var + 1e-5)
        h = (t - mu) * scale + be_ref[0:1, :]
        h = jnp.where(h > 0, h, jnp.exp(jnp.minimum(h, 0.0)) - 1.0)
        if out_parts:
            hb = h.astype(jnp.bfloat16)
            for q in range(2):
                out_ref[q] = hb[:, q * 256:(q + 1) * 256]
        else:
            out_ref[...] = h

    if out_parts:
        out_spec = pl.BlockSpec((2, rblk, 256), lambda i: (0, i, 0))
        out_shape = jax.ShapeDtypeStruct((2, n_nodes, 256), jnp.bfloat16)
    else:
        out_spec = pl.BlockSpec((rblk, 512), lambda i: (i, 0))
        out_shape = jax.ShapeDtypeStruct((n_nodes, 512), jnp.float32)

    return pl.pallas_call(
        body,
        grid=grid,
        in_specs=[
            pl.BlockSpec((rblk, 512), lambda i: (i, 0)),
            pl.BlockSpec((8, 512), lambda i: (0, 0)),
            pl.BlockSpec((1, 512), lambda i: (0, 0)),
            pl.BlockSpec((1, 512), lambda i: (0, 0)),
        ],
        out_specs=out_spec,
        out_shape=out_shape,
    )


def kernel(x, edge_index, W1l, b1l, W1r, g1, be1, W2l, b2l, W2r, g2, be2):
    n, d_in = x.shape
    e = edge_index.shape[1]
    d_h = W1l.shape[0]
    d_out = W2l.shape[0]
    p1 = d_in // 128
    p2 = d_h // 256
    rblk = 1000

    row = edge_index[0]
    col = edge_index[1]

    nb = ((e + 8 * NS * EB - 1) // (8 * NS * EB)) * (8 * NS)
    e_pad = nb * EB
    row_p = jnp.concatenate([row, jnp.full((e_pad - e,), n, jnp.int32)])
    col_p = jnp.concatenate([col, jnp.zeros((e_pad - e,), jnp.int32)])
    rowb = row_p.reshape(nb, 1, EB)
    col2 = (col_p[None, :] + (jnp.arange(2, dtype=jnp.int32) * n)[:, None]
            ).reshape(2, nb, 1, EB)

    x2b = x.astype(jnp.bfloat16).reshape(n, p1, 128).transpose(1, 0, 2)

    w1l_t = W1l.T.reshape(p1, 128, d_h)
    w1r_t = W1r.T.reshape(1, d_in, d_h)
    w2l_t = W2l.T.reshape(p2, 256, d_out)
    w2r_t = W2r.T.reshape(p2, 256, d_out)
    b1 = jnp.broadcast_to(b1l[None, :], (8, d_h))
    b2 = jnp.broadcast_to(b2l[None, :], (8, d_out))
    g1_2 = g1[None, :]
    be1_2 = be1[None, :]
    g2_2 = g2[None, :]
    be2_2 = be2[None, :]

    sc1 = _make_sc_aggregate(n, nb, p1, 128, jnp.bfloat16, with_deg=True)
    agg1, deg = sc1(x2b.reshape(p1 * n, 128), col2, rowb)
    mm1 = _make_tc_matmul(n, p1, 128, 1, d_in, rblk)
    t1, st1 = mm1(agg1, deg, x.reshape(1, n, d_in), w1l_t, w1r_t, b1)
    norm1 = _make_tc_norm(n, rblk, out_parts=True)
    h2b = norm1(t1, st1, g1_2, be1_2)

    sc2 = _make_sc_aggregate(n, nb, p2, 256, jnp.bfloat16, with_deg=False)
    (agg2,) = sc2(h2b.reshape(p2 * n, 256), col2, rowb)
    mm2 = _make_tc_matmul(n, p2, 256, p2, 256, rblk)
    t2, st2 = mm2(agg2, deg, h2b, w2l_t, w2r_t, b2)
    norm2 = _make_tc_norm(n, rblk, out_parts=False)
    out = norm2(t2, st2, g2_2, be2_2)
    return out
